# speedup vs baseline: 1.7110x; 1.7110x over previous
import jax
import jax.numpy as jnp
from jax.experimental import pallas as pl

_N = 256
_D = 256
_H = 128
_BI = 32


def _ab_kernel(emb_ref, w1_ref, a_ref, b_ref):
    emb = emb_ref[...]
    a_ref[...] = jnp.dot(emb, w1_ref[:_D, :], preferred_element_type=jnp.float32)
    b_ref[...] = jnp.dot(emb, w1_ref[_D:, :], preferred_element_type=jnp.float32)


def _ln(x, g, b, eps=1e-5):
    mu = jnp.mean(x, axis=-1, keepdims=True)
    xc = x - mu
    var = jnp.mean(xc * xc, axis=-1, keepdims=True)
    return xc * jax.lax.rsqrt(var + eps) * g + b


def _mlp_kernel(a_ref, b_ref, b1_ref, g1_ref, be1_ref,
                w2_ref, b2_ref, g2_ref, be2_ref,
                w3_ref, b3_ref, vm_ref, out_ref):
    a = a_ref[...]
    b = b_ref[...]
    pre = a[:, None, :] + b[None, :, :] + b1_ref[...][None, :, :]
    pre = pre.reshape(_BI * _N, _H)

    h = jnp.maximum(_ln(pre, g1_ref[...], be1_ref[...]), 0.0)
    h2 = jnp.dot(h, w2_ref[...], preferred_element_type=jnp.float32) + b2_ref[...]
    h2 = jnp.maximum(_ln(h2, g2_ref[...], be2_ref[...]), 0.0)
    s = jnp.dot(h2, w3_ref[...], preferred_element_type=jnp.float32)
    s = s.reshape(_BI, _N) + b3_ref[0, 0]

    i0 = pl.program_id(0) * _BI
    ii = i0 + jax.lax.broadcasted_iota(jnp.int32, (_BI, _N), 0)
    jj = jax.lax.broadcasted_iota(jnp.int32, (_BI, _N), 1)
    offdiag = (ii != jj).astype(jnp.float32)
    out_ref[...] = s * offdiag * vm_ref[...]


@jax.jit
def _run(node_embeddings, valid_mask_f, W1, b1, g1, be1, W2, b2, g2, be2, W3, b3):
    a, b = pl.pallas_call(
        _ab_kernel,
        out_shape=(
            jax.ShapeDtypeStruct((_N, _H), jnp.float32),
            jax.ShapeDtypeStruct((_N, _H), jnp.float32),
        ),
    )(node_embeddings, W1)

    grid = _N // _BI
    out = pl.pallas_call(
        _mlp_kernel,
        grid=(grid,),
        in_specs=[
            pl.BlockSpec((_BI, _H), lambda i: (i, 0)),
            pl.BlockSpec((_N, _H), lambda i: (0, 0)),
            pl.BlockSpec((1, _H), lambda i: (0, 0)),
            pl.BlockSpec((1, _H), lambda i: (0, 0)),
            pl.BlockSpec((1, _H), lambda i: (0, 0)),
            pl.BlockSpec((_H, _H), lambda i: (0, 0)),
            pl.BlockSpec((1, _H), lambda i: (0, 0)),
            pl.BlockSpec((1, _H), lambda i: (0, 0)),
            pl.BlockSpec((1, _H), lambda i: (0, 0)),
            pl.BlockSpec((_H, 1), lambda i: (0, 0)),
            pl.BlockSpec((1, 1), lambda i: (0, 0)),
            pl.BlockSpec((_BI, _N), lambda i: (i, 0)),
        ],
        out_specs=pl.BlockSpec((_BI, _N), lambda i: (i, 0)),
        out_shape=jax.ShapeDtypeStruct((_N, _N), jnp.float32),
    )(a, b,
      b1.reshape(1, _H), g1.reshape(1, _H), be1.reshape(1, _H),
      W2, b2.reshape(1, _H), g2.reshape(1, _H), be2.reshape(1, _H),
      W3, b3.reshape(1, 1), valid_mask_f)
    return out.reshape(_N * _N)


def kernel(node_embeddings, valid_edges, valid_mask, W1, b1, g1, be1, W2, b2, g2, be2, W3, b3):
    del valid_edges
    vm = valid_mask.astype(jnp.float32).reshape(_N, _N)
    return _run(node_embeddings, vm, W1, b1, g1, be1, W2, b2, g2, be2, W3, b3)
